# baseline (speedup 1.0000x reference)
import functools

import jax
import jax.numpy as jnp
from jax import lax
from jax.experimental import pallas as pl
from jax.experimental.pallas import tpu as pltpu
from jax.experimental.pallas import tpu_sc as plsc

N = 10000
E = 160000
D_IN = 512
H = 128
HH = H // 2
G = 64
BN_EPS = 1e-5

N_PAD = 10240
RB = 1024
NS = 16
EPS_ = E // NS
CW = 125
CHUNKS = EPS_ // CW
ROWS_PER_SUB = N_PAD // NS
YROWS_PER_SUB = N // NS


def _dot(a, b):
    return jnp.dot(a, b, preferred_element_type=jnp.float32)


def _mm_body(x_ref, w_ref, l_ref, r_ref):
    y = _dot(x_ref[...], w_ref[...])
    l_ref[...] = y[:, :HH]
    r_ref[...] = y[:, HH:]


def _first_matmul(x, w):
    return pl.pallas_call(
        _mm_body,
        out_shape=(
            jax.ShapeDtypeStruct((N, HH), jnp.float32),
            jax.ShapeDtypeStruct((N, HH), jnp.float32),
        ),
        grid=(N_PAD // RB,),
        in_specs=[
            pl.BlockSpec((RB, D_IN), lambda i: (i, 0)),
            pl.BlockSpec((D_IN, H), lambda i: (0, 0)),
        ],
        out_specs=(
            pl.BlockSpec((RB, HH), lambda i: (i, 0)),
            pl.BlockSpec((RB, HH), lambda i: (i, 0)),
        ),
    )(x, w)


@functools.cache
def _build_edge_agg():
    mesh = plsc.VectorSubcoreMesh(core_axis_name="c", subcore_axis_name="s",
                                  num_cores=2, num_subcores=16)
    return pl.kernel(
        _edge_agg_body,
        out_type=jax.ShapeDtypeStruct((2, N_PAD, HH), jnp.float32),
        mesh=mesh,
        compiler_params=pltpu.CompilerParams(use_tc_tiling_on_sc=False),
        scratch_types=[
            pltpu.VMEM((CHUNKS, CW), jnp.int32),
            pltpu.VMEM((CHUNKS, CW), jnp.int32),
            pltpu.VMEM((CW, HH), jnp.float32),
            pltpu.VMEM((CW, HH), jnp.float32),
            pltpu.VMEM((16, HH), jnp.float32),
            pltpu.VMEM_SHARED((N_PAD, HH), jnp.float32),
            pltpu.SemaphoreType.DMA,
            pltpu.SemaphoreType.DMA,
        ],
    )


def _gather_scatter_loop(y_hbm, src_v, dst_v, rows_a, rows_b, acc_sh,
                         sem_a, sem_b):
    pltpu.async_copy(y_hbm.at[src_v.at[0]], rows_a, sem_a)

    def _body(k, carry):
        j0 = 2 * k
        pltpu.async_copy(y_hbm.at[src_v.at[j0 + 1]], rows_b, sem_b)
        pltpu.make_async_copy(y_hbm.at[src_v.at[j0]], rows_a, sem_a).wait()
        pltpu.sync_copy(rows_a, acc_sh.at[dst_v.at[j0]], add=True)

        @pl.when(k < CHUNKS // 2 - 1)
        def _():
            pltpu.async_copy(y_hbm.at[src_v.at[j0 + 2]], rows_a, sem_a)

        pltpu.make_async_copy(y_hbm.at[src_v.at[j0 + 1]], rows_b,
                              sem_b).wait()
        pltpu.sync_copy(rows_b, acc_sh.at[dst_v.at[j0 + 1]], add=True)
        return carry

    lax.fori_loop(0, CHUNKS // 2, _body, 0)


def _edge_agg_body(yl_hbm, yr_hbm, edge_hbm, out_hbm, src_v, dst_v, rows_a,
                   rows_b, zbuf, acc_sh, sem_a, sem_b):
    c = lax.axis_index("c")
    s = lax.axis_index("s")

    zeros16 = jnp.zeros((16,), jnp.float32)

    def _zb(i, carry):
        zbuf[i // (HH // 16), pl.ds((i % (HH // 16)) * 16, 16)] = zeros16
        return carry

    lax.fori_loop(0, 16 * (HH // 16), _zb, 0)

    def _zacc(k, carry):
        pltpu.sync_copy(zbuf, acc_sh.at[pl.ds(s * ROWS_PER_SUB + k * 16, 16)])
        return carry

    lax.fori_loop(0, ROWS_PER_SUB // 16, _zacc, 0)

    pltpu.sync_copy(edge_hbm.at[0, s], src_v)
    pltpu.sync_copy(edge_hbm.at[1, s], dst_v)

    plsc.subcore_barrier()

    @pl.when(c == 0)
    def _():
        _gather_scatter_loop(yl_hbm, src_v, dst_v, rows_a, rows_b, acc_sh,
                             sem_a, sem_b)

    @pl.when(c == 1)
    def _():
        _gather_scatter_loop(yr_hbm, src_v, dst_v, rows_a, rows_b, acc_sh,
                             sem_a, sem_b)

    plsc.subcore_barrier()

    pltpu.sync_copy(
        acc_sh.at[pl.ds(s * ROWS_PER_SUB, ROWS_PER_SUB)],
        out_hbm.at[c, pl.ds(s * ROWS_PER_SUB, ROWS_PER_SUB)],
    )


def _bn_mlp(yl_ref, yr_ref, a_ref, sc_ref, bi_ref, w2_ref, b2_ref):
    y = jnp.concatenate([yl_ref[...], yr_ref[...]], axis=1)
    a = jnp.concatenate([a_ref[0], a_ref[1]], axis=1)
    z = (y + a) * sc_ref[...] + bi_ref[...]
    z = jnp.maximum(z, 0.0)
    return jnp.maximum(_dot(z, w2_ref[...]) + b2_ref[...], 0.0)


def _pool_update(i, bt_ref, h, pool_ref):
    @pl.when(i == 0)
    def _():
        pool_ref[...] = jnp.zeros_like(pool_ref)

    rvalid = (lax.broadcasted_iota(jnp.int32, (RB, 1), 0) + i * RB) < N
    h = jnp.where(rvalid, h, 0.0)
    seg = lax.broadcasted_iota(jnp.int32, (G, RB), 0)
    mask = (seg == bt_ref[...][None, :]).astype(jnp.float32)
    pool_ref[...] += _dot(mask, h)


def _mlp_body_next(yl_ref, yr_ref, a_ref, sc_ref, bi_ref, w2_ref, b2_ref,
                   wn_ref, bt_ref, ynl_ref, ynr_ref, pool_ref):
    i = pl.program_id(0)
    h = _bn_mlp(yl_ref, yr_ref, a_ref, sc_ref, bi_ref, w2_ref, b2_ref)
    yn = _dot(h, wn_ref[...])
    ynl_ref[...] = yn[:, :HH]
    ynr_ref[...] = yn[:, HH:]
    _pool_update(i, bt_ref, h, pool_ref)


def _mlp_body_last(yl_ref, yr_ref, a_ref, sc_ref, bi_ref, w2_ref, b2_ref,
                   bt_ref, p1_ref, p2_ref, l1w_ref, l1b_ref, l2w_ref,
                   l2b_ref, lo_ref, ls_ref, pool_ref):
    i = pl.program_id(0)
    h = _bn_mlp(yl_ref, yr_ref, a_ref, sc_ref, bi_ref, w2_ref, b2_ref)
    _pool_update(i, bt_ref, h, pool_ref)

    @pl.when(i == pl.num_programs(0) - 1)
    def _():
        q = (_dot(p1_ref[...], l1w_ref[0:H, :])
             + _dot(p2_ref[...], l1w_ref[H:2 * H, :])
             + _dot(pool_ref[...], l1w_ref[2 * H:3 * H, :]))
        q = jnp.maximum(q + l1b_ref[...], 0.0)
        logits = _dot(q, l2w_ref[...]) + l2b_ref[...]
        m = jnp.max(logits, axis=1, keepdims=True)
        ls = logits - m
        ls = ls - jnp.log(jnp.sum(jnp.exp(ls), axis=1, keepdims=True))
        lo_ref[...] = logits
        ls_ref[...] = ls


def _mlp_stage(yl, yr, acc, vscale, vbias, w2, b2, batch, wn, head_args=None):
    grid = (N_PAD // RB,)
    half_spec = pl.BlockSpec((RB, HH), lambda i: (i, 0))
    acc_spec = pl.BlockSpec((2, RB, HH), lambda i: (0, i, 0))
    vec_spec = pl.BlockSpec((1, H), lambda i: (0, 0))
    w_spec = pl.BlockSpec((H, H), lambda i: (0, 0))
    bt_spec = pl.BlockSpec((RB,), lambda i: (i,))
    pool_spec = pl.BlockSpec((G, H), lambda i: (0, 0))

    if wn is not None:
        return pl.pallas_call(
            _mlp_body_next,
            out_shape=(
                jax.ShapeDtypeStruct((N, HH), jnp.float32),
                jax.ShapeDtypeStruct((N, HH), jnp.float32),
                jax.ShapeDtypeStruct((G, H), jnp.float32),
            ),
            grid=grid,
            in_specs=[half_spec, half_spec, acc_spec, vec_spec, vec_spec,
                      w_spec, vec_spec, w_spec, bt_spec],
            out_specs=(half_spec, half_spec, pool_spec),
        )(yl, yr, acc, vscale, vbias, w2, b2, wn, batch)
    p1, p2, l1_W, l1_b, l2_W, l2_b = head_args
    small = lambda shape: pl.BlockSpec(shape, lambda i: (0,) * len(shape))
    return pl.pallas_call(
        _mlp_body_last,
        out_shape=(
            jax.ShapeDtypeStruct((G, 2), jnp.float32),
            jax.ShapeDtypeStruct((G, 2), jnp.float32),
            jax.ShapeDtypeStruct((G, H), jnp.float32),
        ),
        grid=grid,
        in_specs=[half_spec, half_spec, acc_spec, vec_spec, vec_spec,
                  w_spec, vec_spec, bt_spec, small((G, H)), small((G, H)),
                  small((3 * H, 64)), small((1, 64)), small((64, 2)),
                  small((1, 2))],
        out_specs=(small((G, 2)), small((G, 2)), pool_spec),
    )(yl, yr, acc, vscale, vbias, w2, b2, batch, p1, p2, l1_W,
      l1_b.reshape(1, 64), l2_W, l2_b.reshape(1, 2))


def kernel(x, edge_index, batch,
           c1_W1, c1_b1, c1_g, c1_be, c1_W2, c1_b2,
           c2_W1, c2_b1, c2_g, c2_be, c2_W2, c2_b2,
           c3_W1, c3_b1, c3_g, c3_be, c3_W2, c3_b2,
           l1_W, l1_b, l2_W, l2_b):
    e4 = edge_index.reshape(2, NS, CHUNKS, CW)

    bn = 1.0 / jnp.sqrt(jnp.float32(1.0 + BN_EPS))
    params = []
    for (b1, g, be, w2, b2, wn) in (
        (c1_b1, c1_g, c1_be, c1_W2, c1_b2, c2_W1),
        (c2_b1, c2_g, c2_be, c2_W2, c2_b2, c3_W1),
        (c3_b1, c3_g, c3_be, c3_W2, c3_b2, None),
    ):
        vscale = (g * bn).reshape(1, H)
        vbias = (b1 * g * bn + be).reshape(1, H)
        params.append((vscale, vbias, w2, b2.reshape(1, H), wn))

    yl, yr = _first_matmul(x, c1_W1)
    pools = []
    for (vscale, vbias, w2, b2, wn) in params:
        acc = _build_edge_agg()(yl, yr, e4)
        if wn is not None:
            yl, yr, pool = _mlp_stage(yl, yr, acc, vscale, vbias, w2, b2,
                                      batch, wn)
            pools.append(pool)
        else:
            head_args = (pools[0], pools[1], l1_W, l1_b, l2_W, l2_b)
            logits, ls, _ = _mlp_stage(yl, yr, acc, vscale, vbias, w2, b2,
                                       batch, None, head_args)
    return (logits, ls)


# speedup vs baseline: 1.3147x; 1.3147x over previous
import functools

import jax
import jax.numpy as jnp
from jax import lax
from jax.experimental import pallas as pl
from jax.experimental.pallas import tpu as pltpu
from jax.experimental.pallas import tpu_sc as plsc

N = 10000
E = 160000
D_IN = 512
H = 128
G = 64
BN_EPS = 1e-5

N_PAD = 10240
RB = 1024
NW = 32
EPW = E // NW
CW = 125
CHUNKS = EPW // CW
ROWS_PER_SUB = N_PAD // 16

def _dot(a, b):
    return jnp.dot(a, b, preferred_element_type=jnp.float32)


def _mm_body(x_ref, w_ref, o_ref):
    o_ref[...] = _dot(x_ref[...], w_ref[...])


def _first_matmul(x, w):
    return pl.pallas_call(
        _mm_body,
        out_shape=jax.ShapeDtypeStruct((N, H), jnp.float32),
        grid=(N_PAD // 1024,),
        in_specs=[
            pl.BlockSpec((1024, D_IN), lambda i: (i, 0)),
            pl.BlockSpec((D_IN, H), lambda i: (0, 0)),
        ],
        out_specs=pl.BlockSpec((1024, H), lambda i: (i, 0)),
    )(x, w)


@functools.cache
def _build_edge_agg():
    mesh = plsc.VectorSubcoreMesh(core_axis_name="c", subcore_axis_name="s",
                                  num_cores=2, num_subcores=16)
    return pl.kernel(
        _edge_agg_body,
        out_type=jax.ShapeDtypeStruct((2 * N_PAD, H), jnp.float32),
        mesh=mesh,
        scratch_types=[
            pltpu.VMEM((CHUNKS, CW), jnp.int32),
            pltpu.VMEM((CHUNKS, CW), jnp.int32),
            pltpu.VMEM((CW, H), jnp.float32),
            pltpu.VMEM((CW, H), jnp.float32),
            pltpu.VMEM((16, H), jnp.float32),
            pltpu.VMEM_SHARED((N_PAD, H), jnp.float32),
            pltpu.SemaphoreType.DMA,
            pltpu.SemaphoreType.DMA,
        ],
    )


def _edge_agg_body(y_hbm, edge_hbm, out_hbm, src_v, dst_v, rows_a,
                   rows_b, zbuf, acc_sh, sem_a, sem_b):
    c = lax.axis_index("c")
    s = lax.axis_index("s")
    wid = c * 16 + s

    pltpu.sync_copy(edge_hbm.at[0, wid], src_v)
    pltpu.async_copy(y_hbm.at[src_v.at[0]], rows_a, sem_a)
    pltpu.async_copy(y_hbm.at[src_v.at[1]], rows_b, sem_b)
    pltpu.sync_copy(edge_hbm.at[1, wid], dst_v)

    zeros16 = jnp.zeros((16,), jnp.float32)

    def _zb(i, carry):
        zbuf[i // 8, pl.ds((i % 8) * 16, 16)] = zeros16
        return carry

    lax.fori_loop(0, 16 * (H // 16), _zb, 0)

    def _zacc(k, carry):
        pltpu.sync_copy(zbuf, acc_sh.at[pl.ds(s * ROWS_PER_SUB + k * 16, 16)])
        return carry

    lax.fori_loop(0, ROWS_PER_SUB // 16, _zacc, 0)

    plsc.subcore_barrier()

    def _body(k, carry):
        j0 = 2 * k
        pltpu.make_async_copy(y_hbm.at[src_v.at[j0]], rows_a, sem_a).wait()
        pltpu.sync_copy(rows_a, acc_sh.at[dst_v.at[j0]], add=True)

        @pl.when(k < CHUNKS // 2 - 1)
        def _():
            pltpu.async_copy(y_hbm.at[src_v.at[j0 + 2]], rows_a, sem_a)

        pltpu.make_async_copy(y_hbm.at[src_v.at[j0 + 1]], rows_b,
                              sem_b).wait()
        pltpu.sync_copy(rows_b, acc_sh.at[dst_v.at[j0 + 1]], add=True)

        @pl.when(k < CHUNKS // 2 - 1)
        def _():
            pltpu.async_copy(y_hbm.at[src_v.at[j0 + 3]], rows_b, sem_b)
        return carry

    lax.fori_loop(0, CHUNKS // 2, _body, 0)

    plsc.subcore_barrier()

    pltpu.sync_copy(
        acc_sh.at[pl.ds(s * ROWS_PER_SUB, ROWS_PER_SUB)],
        out_hbm.at[pl.ds(c * N_PAD + s * ROWS_PER_SUB, ROWS_PER_SUB)],
    )


def _mlp_body_next(y_ref, a0_ref, a1_ref, sc_ref, bi_ref, w2_ref, b2_ref,
                   wn_ref, bt_ref, yn_ref, pool_ref):
    i = pl.program_id(0)
    z = y_ref[...] + a0_ref[...] + a1_ref[...]
    z = z * sc_ref[...] + bi_ref[...]
    z = jnp.maximum(z, 0.0)
    h = jnp.maximum(_dot(z, w2_ref[...]) + b2_ref[...], 0.0)
    yn_ref[...] = _dot(h, wn_ref[...])

    @pl.when(i == 0)
    def _():
        pool_ref[...] = jnp.zeros_like(pool_ref)

    rvalid = (lax.broadcasted_iota(jnp.int32, (RB, 1), 0) + i * RB) < N
    h = jnp.where(rvalid, h, 0.0)
    seg = lax.broadcasted_iota(jnp.int32, (G, RB), 0)
    mask = (seg == bt_ref[...][None, :]).astype(jnp.float32)
    pool_ref[...] += _dot(mask, h)


def _mlp_body_last(y_ref, a0_ref, a1_ref, sc_ref, bi_ref, w2_ref, b2_ref,
                   bt_ref, p1_ref, p2_ref, l1w_ref, l1b_ref, l2w_ref,
                   l2b_ref, lo_ref, ls_ref, pool_ref):
    i = pl.program_id(0)
    z = y_ref[...] + a0_ref[...] + a1_ref[...]
    z = z * sc_ref[...] + bi_ref[...]
    z = jnp.maximum(z, 0.0)
    h = jnp.maximum(_dot(z, w2_ref[...]) + b2_ref[...], 0.0)

    @pl.when(i == 0)
    def _():
        pool_ref[...] = jnp.zeros_like(pool_ref)

    rvalid = (lax.broadcasted_iota(jnp.int32, (RB, 1), 0) + i * RB) < N
    h = jnp.where(rvalid, h, 0.0)
    seg = lax.broadcasted_iota(jnp.int32, (G, RB), 0)
    mask = (seg == bt_ref[...][None, :]).astype(jnp.float32)
    pool_ref[...] += _dot(mask, h)

    @pl.when(i == pl.num_programs(0) - 1)
    def _():
        q = (_dot(p1_ref[...], l1w_ref[0:H, :])
             + _dot(p2_ref[...], l1w_ref[H:2 * H, :])
             + _dot(pool_ref[...], l1w_ref[2 * H:3 * H, :]))
        q = jnp.maximum(q + l1b_ref[...], 0.0)
        logits = _dot(q, l2w_ref[...]) + l2b_ref[...]
        m = jnp.max(logits, axis=1, keepdims=True)
        ls = logits - m
        ls = ls - jnp.log(jnp.sum(jnp.exp(ls), axis=1, keepdims=True))
        lo_ref[...] = logits
        ls_ref[...] = ls


def _mlp_stage(y, acc, vscale, vbias, w2, b2, batch_pad, wn, head_args=None):
    grid = (N_PAD // RB,)
    row_spec = pl.BlockSpec((RB, H), lambda i: (i, 0))
    acc0_spec = pl.BlockSpec((RB, H), lambda i: (i, 0))
    acc1_spec = pl.BlockSpec((RB, H), lambda i: (i + N_PAD // RB, 0))
    vec_spec = pl.BlockSpec((1, H), lambda i: (0, 0))
    w_spec = pl.BlockSpec((H, H), lambda i: (0, 0))
    bt_spec = pl.BlockSpec((RB,), lambda i: (i,))
    pool_spec = pl.BlockSpec((G, H), lambda i: (0, 0))

    if wn is not None:
        return pl.pallas_call(
            _mlp_body_next,
            out_shape=(
                jax.ShapeDtypeStruct((N, H), jnp.float32),
                jax.ShapeDtypeStruct((G, H), jnp.float32),
            ),
            grid=grid,
            in_specs=[row_spec, acc0_spec, acc1_spec, vec_spec, vec_spec,
                      w_spec, vec_spec, w_spec, bt_spec],
            out_specs=(row_spec, pool_spec),
        )(y, acc, acc, vscale, vbias, w2, b2, wn, batch_pad)
    p1, p2, l1_W, l1_b, l2_W, l2_b = head_args
    small = lambda shape: pl.BlockSpec(shape, lambda i: (0,) * len(shape))
    return pl.pallas_call(
        _mlp_body_last,
        out_shape=(
            jax.ShapeDtypeStruct((G, 2), jnp.float32),
            jax.ShapeDtypeStruct((G, 2), jnp.float32),
            jax.ShapeDtypeStruct((G, H), jnp.float32),
        ),
        grid=grid,
        in_specs=[row_spec, acc0_spec, acc1_spec, vec_spec, vec_spec,
                  w_spec, vec_spec, bt_spec, small((G, H)), small((G, H)),
                  small((3 * H, 64)), small((1, 64)), small((64, 2)),
                  small((1, 2))],
        out_specs=(small((G, 2)), small((G, 2)), pool_spec),
    )(y, acc, acc, vscale, vbias, w2, b2, batch_pad, p1, p2, l1_W,
      l1_b.reshape(1, 64), l2_W, l2_b.reshape(1, 2))


def kernel(x, edge_index, batch,
           c1_W1, c1_b1, c1_g, c1_be, c1_W2, c1_b2,
           c2_W1, c2_b1, c2_g, c2_be, c2_W2, c2_b2,
           c3_W1, c3_b1, c3_g, c3_be, c3_W2, c3_b2,
           l1_W, l1_b, l2_W, l2_b):
    e4 = edge_index.reshape(2, NW, CHUNKS, CW)

    bn = 1.0 / jnp.sqrt(jnp.float32(1.0 + BN_EPS))
    params = []
    for (b1, g, be, w2, b2, wn) in (
        (c1_b1, c1_g, c1_be, c1_W2, c1_b2, c2_W1),
        (c2_b1, c2_g, c2_be, c2_W2, c2_b2, c3_W1),
        (c3_b1, c3_g, c3_be, c3_W2, c3_b2, None),
    ):
        vscale = (g * bn).reshape(1, H)
        vbias = (b1 * g * bn + be).reshape(1, H)
        params.append((vscale, vbias, w2, b2.reshape(1, H), wn))

    y = _first_matmul(x, c1_W1)
    pools = []
    for (vscale, vbias, w2, b2, wn) in params:
        acc = _build_edge_agg()(y, e4)
        if wn is not None:
            y, pool = _mlp_stage(y, acc, vscale, vbias, w2, b2, batch, wn)
            pools.append(pool)
        else:
            head_args = (pools[0], pools[1], l1_W, l1_b, l2_W, l2_b)
            logits, ls, _ = _mlp_stage(y, acc, vscale, vbias, w2, b2,
                                       batch, None, head_args)
    return (logits, ls)
